# baseline (speedup 1.0000x reference)
import jax
import jax.numpy as jnp
from jax.experimental import pallas as pl

_B = 16384
_F = 26
_D = 64
_NFLAT = _B * _F
_BB = 128


def _mm_body(e_ref, w_ref, b_ref, o_ref):
  acc = jnp.dot(e_ref[...], w_ref[...], preferred_element_type=jnp.float32)
  y = jnp.maximum(acc + b_ref[0:1, :], 0.0)
  o_ref[...] = y.reshape(_BB, _F, _D)


def _tc_linear_relu(e, W, b):
  b2 = jnp.broadcast_to(b[None, :], (8, _D))
  grid = (_B // _BB,)
  return pl.pallas_call(
      _mm_body,
      grid=grid,
      in_specs=[
          pl.BlockSpec((_BB * _F, _D), lambda i: (i, 0)),
          pl.BlockSpec((_D, _D), lambda i: (0, 0)),
          pl.BlockSpec((8, _D), lambda i: (0, 0)),
      ],
      out_specs=pl.BlockSpec((_BB, _F, _D), lambda i: (i, 0, 0)),
      out_shape=jax.ShapeDtypeStruct((_B, _F, _D), jnp.float32),
  )(e, W, b2)


def kernel(x, table, W, b):
  e = jnp.take(table, x.reshape(_NFLAT), axis=0)
  return _tc_linear_relu(e, W, b)
